# Initial kernel scaffold; baseline (speedup 1.0000x reference)
import jax
import jax.numpy as jnp
from jax.experimental import pallas as pl


def kernel(x, Wg, bg, W1, b1, W2, b2):
    raise NotImplementedError("write your pallas kernel here")


# speedup vs baseline: 2.8116x; 2.8116x over previous
import functools

import jax
import jax.numpy as jnp
from jax.experimental import pallas as pl
from jax.experimental.pallas import tpu as pltpu

B, S, D = 1, 2048, 1024
E, K, F = 8, 2, 2048

BLK = 128
CAP = S * K + E * BLK
NB = CAP // BLK
PCH = 512
CBLK = 128
RCH = 128


def _routing_body(x_ref, wg_ref, bg_ref, perm_ref, posab_ref, wab_ref,
                  be_ref, ptot_ref):
    x = x_ref[...]
    logits = jnp.dot(x, wg_ref[...], preferred_element_type=jnp.float32)
    logits = logits + bg_ref[...]
    lm = jnp.max(logits, axis=1, keepdims=True)
    p = jnp.exp(logits - lm)
    sm = p / jnp.sum(p, axis=1, keepdims=True)
    idx8 = jax.lax.broadcasted_iota(jnp.int32, (S, E), 1)
    m1 = jnp.max(sm, axis=1, keepdims=True)
    e1 = jnp.min(jnp.where(sm == m1, idx8, E), axis=1, keepdims=True)
    h1 = idx8 == e1
    sm2 = jnp.where(h1, -jnp.inf, sm)
    m2 = jnp.max(sm2, axis=1, keepdims=True)
    e2 = jnp.min(jnp.where(sm2 == m2, idx8, E), axis=1, keepdims=True)
    h2 = idx8 == e2
    w1 = m1 / (m1 + m2)
    w2 = m2 / (m1 + m2)
    assign = (h1 | h2).astype(jnp.float32)

    ranks = []
    for i in range(S // RCH):
        row = jax.lax.broadcasted_iota(jnp.int32, (RCH, S), 0) + i * RCH
        col = jax.lax.broadcasted_iota(jnp.int32, (RCH, S), 1)
        m = (col < row).astype(jnp.float32)
        ranks.append(jnp.dot(m, assign, preferred_element_type=jnp.float32))
    rank = jnp.concatenate(ranks, axis=0)

    counts = jnp.sum(assign, axis=0, keepdims=True)
    ci = counts.astype(jnp.int32)
    pci = jnp.bitwise_and(ci + (BLK - 1), ~(BLK - 1))
    cols = []
    acc = jnp.zeros((1, 1), jnp.int32)
    for e in range(E):
        cols.append(acc)
        acc = acc + pci[0:1, e:e + 1]
    base = jnp.concatenate(cols, axis=1)
    ptot_ref[...] = acc

    pos = base.astype(jnp.float32) + rank
    pos_a = jnp.sum(jnp.where(h1, pos, 0.0), axis=1, keepdims=True)
    pos_b = jnp.sum(jnp.where(h2, pos, 0.0), axis=1, keepdims=True)
    posab_ref[...] = jnp.concatenate(
        [pos_a, pos_b], axis=1).astype(jnp.int32)
    wab_ref[...] = jnp.concatenate([w1, w2], axis=1)

    tokf = jax.lax.broadcasted_iota(jnp.int32, (S, 1), 0).astype(jnp.float32)
    for c in range(CAP // PCH):
        piota = (jax.lax.broadcasted_iota(jnp.int32, (1, PCH), 1)
                 + c * PCH).astype(jnp.float32)
        eq_a = pos_a == piota
        eq_b = pos_b == piota
        contrib = jnp.where(eq_a, tokf, 0.0) + jnp.where(eq_b, tokf, 0.0)
        permc = jnp.sum(contrib, axis=0, keepdims=True)
        perm_ref[0:1, c * PCH:(c + 1) * PCH] = permc.astype(jnp.int32)

    iob = jax.lax.broadcasted_iota(jnp.int32, (1, NB), 1) * BLK
    bev = jnp.zeros((1, NB), jnp.int32)
    for e in range(E):
        end_e = base[0:1, e:e + 1] + pci[0:1, e:e + 1]
        bev = bev + (end_e <= iob).astype(jnp.int32)
    be_ref[...] = jnp.minimum(bev, E - 1)


def _ffn_body(be_ref, perm_ref, ptot_ref, x_ref, w1_ref, b1_ref, w2_ref,
              b2_ref, out_ref, xs_ref):
    i = pl.program_id(0)

    @pl.when(i * BLK < ptot_ref[0])
    def _():
        def gather_row(r, carry):
            t = perm_ref[i * BLK + r]
            xs_ref[pl.ds(r, 1), :] = x_ref[pl.ds(t, 1), :]
            return carry
        jax.lax.fori_loop(0, BLK, gather_row, 0)
        xs = xs_ref[...]
        h = jnp.dot(xs, w1_ref[0], preferred_element_type=jnp.float32)
        h = h + b1_ref[0]
        h = 0.5 * h * (1.0 + jax.lax.erf(h * (2.0 ** -0.5)))
        y = jnp.dot(h, w2_ref[0], preferred_element_type=jnp.float32)
        out_ref[...] = y + b2_ref[0]


def _combine_body(pa_ref, pb_ref, wa_ref, wb_ref, ys_ref, out_ref):
    i = pl.program_id(0)

    def row(r, carry):
        t = i * CBLK + r
        pa = pa_ref[t]
        pb = pb_ref[t]
        out_ref[pl.ds(r, 1), :] = (ys_ref[pl.ds(pa, 1), :] * wa_ref[t] +
                                   ys_ref[pl.ds(pb, 1), :] * wb_ref[t])
        return carry
    jax.lax.fori_loop(0, CBLK, row, 0)


@jax.jit
def _moe(x2, Wg, bg2, W1, b13, W2, b23):
    perm, posab, wab, be, ptot = pl.pallas_call(
        _routing_body,
        out_shape=[
            jax.ShapeDtypeStruct((1, CAP), jnp.int32),
            jax.ShapeDtypeStruct((S, 2), jnp.int32),
            jax.ShapeDtypeStruct((S, 2), jnp.float32),
            jax.ShapeDtypeStruct((1, NB), jnp.int32),
            jax.ShapeDtypeStruct((1, 1), jnp.int32),
        ],
    )(x2, Wg, bg2)

    ys = pl.pallas_call(
        _ffn_body,
        grid_spec=pltpu.PrefetchScalarGridSpec(
            num_scalar_prefetch=3,
            grid=(NB,),
            in_specs=[
                pl.BlockSpec((S, D), lambda i, be, pm, pt: (0, 0)),
                pl.BlockSpec((1, D, F), lambda i, be, pm, pt: (be[i], 0, 0)),
                pl.BlockSpec((1, 1, F), lambda i, be, pm, pt: (be[i], 0, 0)),
                pl.BlockSpec((1, F, D), lambda i, be, pm, pt: (be[i], 0, 0)),
                pl.BlockSpec((1, 1, D), lambda i, be, pm, pt: (be[i], 0, 0)),
            ],
            out_specs=pl.BlockSpec((BLK, D), lambda i, be, pm, pt: (i, 0)),
            scratch_shapes=[pltpu.VMEM((BLK, D), jnp.float32)],
        ),
        out_shape=jax.ShapeDtypeStruct((CAP, D), jnp.float32),
    )(be.reshape(NB), perm.reshape(CAP), ptot.reshape(1),
      x2, W1, b13, W2, b23)

    out = pl.pallas_call(
        _combine_body,
        grid_spec=pltpu.PrefetchScalarGridSpec(
            num_scalar_prefetch=4,
            grid=(S // CBLK,),
            in_specs=[pl.BlockSpec((CAP, D), lambda i, *_: (0, 0))],
            out_specs=pl.BlockSpec((CBLK, D), lambda i, *_: (i, 0)),
        ),
        out_shape=jax.ShapeDtypeStruct((S, D), jnp.float32),
    )(posab[:, 0], posab[:, 1], wab[:, 0], wab[:, 1], ys)
    return out


def kernel(x, Wg, bg, W1, b1, W2, b2):
    out = _moe(x.reshape(S, D), Wg, bg.reshape(1, E), W1,
               b1.reshape(E, 1, F), W2, b2.reshape(E, 1, D))
    return out.reshape(B, S, D)


# speedup vs baseline: 4.4134x; 1.5697x over previous
import functools

import jax
import jax.numpy as jnp
from jax.experimental import pallas as pl
from jax.experimental.pallas import tpu as pltpu

B, S, D = 1, 2048, 1024
E, K, F = 8, 2, 2048

BLK = 128
CAP = S * K + E * BLK
NB = CAP // BLK
PCH = 512
CBLK = 128
RCH = 128


def _routing_body(x_ref, wg_ref, bg_ref, perm_ref, posab_ref, wab_ref,
                  be_ref, rid_ref, nxe_ref, ptot_ref):
    x = x_ref[...]
    logits = jnp.dot(x, wg_ref[...], preferred_element_type=jnp.float32)
    logits = logits + bg_ref[...]
    lm = jnp.max(logits, axis=1, keepdims=True)
    p = jnp.exp(logits - lm)
    sm = p / jnp.sum(p, axis=1, keepdims=True)
    idx8 = jax.lax.broadcasted_iota(jnp.int32, (S, E), 1)
    m1 = jnp.max(sm, axis=1, keepdims=True)
    e1 = jnp.min(jnp.where(sm == m1, idx8, E), axis=1, keepdims=True)
    h1 = idx8 == e1
    sm2 = jnp.where(h1, -jnp.inf, sm)
    m2 = jnp.max(sm2, axis=1, keepdims=True)
    e2 = jnp.min(jnp.where(sm2 == m2, idx8, E), axis=1, keepdims=True)
    h2 = idx8 == e2
    w1 = m1 / (m1 + m2)
    w2 = m2 / (m1 + m2)
    assign = (h1 | h2).astype(jnp.float32)

    ranks = []
    for i in range(S // RCH):
        row = jax.lax.broadcasted_iota(jnp.int32, (RCH, S), 0) + i * RCH
        col = jax.lax.broadcasted_iota(jnp.int32, (RCH, S), 1)
        m = (col < row).astype(jnp.float32)
        ranks.append(jnp.dot(m, assign, preferred_element_type=jnp.float32))
    rank = jnp.concatenate(ranks, axis=0)

    counts = jnp.sum(assign, axis=0, keepdims=True)
    ci = counts.astype(jnp.int32)
    pci = jnp.bitwise_and(ci + (BLK - 1), ~(BLK - 1))
    cols = []
    acc = jnp.zeros((1, 1), jnp.int32)
    for e in range(E):
        cols.append(acc)
        acc = acc + pci[0:1, e:e + 1]
    base = jnp.concatenate(cols, axis=1)
    ptot_ref[...] = acc

    pos = base.astype(jnp.float32) + rank
    pos_a = jnp.sum(jnp.where(h1, pos, 0.0), axis=1, keepdims=True)
    pos_b = jnp.sum(jnp.where(h2, pos, 0.0), axis=1, keepdims=True)
    posab_ref[...] = jnp.concatenate(
        [pos_a, pos_b], axis=1).astype(jnp.int32)
    wab_ref[...] = jnp.concatenate([w1, w2], axis=1)

    tokf = jax.lax.broadcasted_iota(jnp.int32, (S, 1), 0).astype(jnp.float32)
    for c in range(CAP // PCH):
        piota = (jax.lax.broadcasted_iota(jnp.int32, (1, PCH), 1)
                 + c * PCH).astype(jnp.float32)
        eq_a = pos_a == piota
        eq_b = pos_b == piota
        contrib = jnp.where(eq_a, tokf, 0.0) + jnp.where(eq_b, tokf, 0.0)
        permc = jnp.sum(contrib, axis=0, keepdims=True)
        perm_ref[0:1, c * PCH:(c + 1) * PCH] = permc.astype(jnp.int32)

    iob = jax.lax.broadcasted_iota(jnp.int32, (1, NB), 1) * BLK
    bev = jnp.zeros((1, NB), jnp.int32)
    for e in range(E):
        end_e = base[0:1, e:e + 1] + pci[0:1, e:e + 1]
        bev = bev + (end_e <= iob).astype(jnp.int32)
    bev = jnp.minimum(bev, E - 1)
    be_ref[...] = bev

    shifted = jnp.concatenate(
        [jnp.full((1, 1), -1, jnp.int32), bev[:, :NB - 1]], axis=1)
    trans = (bev != shifted).astype(jnp.float32)
    ublk = jax.lax.broadcasted_iota(jnp.int32, (NB, NB), 0)
    lblk = jax.lax.broadcasted_iota(jnp.int32, (NB, NB), 1)
    ltri = (ublk <= lblk).astype(jnp.float32)
    rid = (jnp.dot(trans, ltri, preferred_element_type=jnp.float32)
           .astype(jnp.int32) - 1)
    transi = trans.astype(jnp.int32)
    nruns = rid[0:1, NB - 1:NB] + 1
    blkvalid = (iob < acc)
    run_expert = []
    run_valid = []
    for r in range(E):
        sel = (rid == r) & (transi == 1)
        run_expert.append(jnp.sum(jnp.where(sel, bev, 0), axis=1,
                                  keepdims=True))
        run_valid.append(jnp.sum(jnp.where(sel & blkvalid, 1, 0), axis=1,
                                 keepdims=True))
    nxe = jnp.full((1, NB), -1, jnp.int32)
    for r in range(E - 1):
        nxt = jnp.where(run_valid[r + 1] > 0, run_expert[r + 1], -1)
        nxe = jnp.where(rid == r, nxt, nxe)
    rid_ref[...] = rid
    nxe_ref[...] = nxe


def _w_copy(w1_any, w2_any, w1b_ref, w2b_ref, sem1, sem2, e, s):
    c1 = pltpu.make_async_copy(w1_any.at[e], w1b_ref.at[s], sem1.at[s])
    c2 = pltpu.make_async_copy(w2_any.at[e], w2b_ref.at[s], sem2.at[s])
    return c1, c2


def _ffn_body(be_ref, rid_ref, nxe_ref, perm_ref, ptot_ref, x_ref, b1_ref,
              b2_ref, w1_any, w2_any, out_ref, xs_ref, w1b_ref, w2b_ref,
              sem1, sem2):
    i = pl.program_id(0)
    rid = rid_ref[i]
    slot = jax.lax.rem(rid, 2)
    prev_rid = rid_ref[jnp.maximum(i - 1, 0)]
    is_first = jnp.logical_or(i == 0, rid != prev_rid)
    valid = i * BLK < ptot_ref[0]

    @pl.when(jnp.logical_and(i == 0, True))
    def _():
        c1, c2 = _w_copy(w1_any, w2_any, w1b_ref, w2b_ref, sem1, sem2,
                         be_ref[0], 0)
        c1.start()
        c2.start()

    @pl.when(jnp.logical_and(jnp.logical_and(is_first, valid),
                             nxe_ref[i] >= 0))
    def _():
        nslot = jax.lax.rem(rid + 1, 2)
        c1, c2 = _w_copy(w1_any, w2_any, w1b_ref, w2b_ref, sem1, sem2,
                         nxe_ref[i], nslot)
        c1.start()
        c2.start()

    @pl.when(jnp.logical_and(is_first, valid))
    def _():
        c1, c2 = _w_copy(w1_any, w2_any, w1b_ref, w2b_ref, sem1, sem2,
                         be_ref[i], slot)
        c1.wait()
        c2.wait()

    @pl.when(valid)
    def _():
        for r in range(BLK):
            t = perm_ref[i * BLK + r]
            xs_ref[pl.ds(r, 1), :] = x_ref[pl.ds(t, 1), :]
        xs = xs_ref[...]
        w1 = w1b_ref[pl.ds(slot, 1)][0]
        h = jnp.dot(xs, w1, preferred_element_type=jnp.float32)
        h = h + b1_ref[0]
        h = 0.5 * h * (1.0 + jax.lax.erf(h * (2.0 ** -0.5)))
        w2 = w2b_ref[pl.ds(slot, 1)][0]
        y = jnp.dot(h, w2, preferred_element_type=jnp.float32)
        out_ref[...] = y + b2_ref[0]


def _combine_body(pa_ref, pb_ref, wa_ref, wb_ref, ys_ref, out_ref):
    i = pl.program_id(0)

    for r in range(CBLK):
        t = i * CBLK + r
        pa = pa_ref[t]
        pb = pb_ref[t]
        out_ref[pl.ds(r, 1), :] = (ys_ref[pl.ds(pa, 1), :] * wa_ref[t] +
                                   ys_ref[pl.ds(pb, 1), :] * wb_ref[t])


@jax.jit
def _moe(x2, Wg, bg2, W1, b13, W2, b23):
    perm, posab, wab, be, rid, nxe, ptot = pl.pallas_call(
        _routing_body,
        out_shape=[
            jax.ShapeDtypeStruct((1, CAP), jnp.int32),
            jax.ShapeDtypeStruct((S, 2), jnp.int32),
            jax.ShapeDtypeStruct((S, 2), jnp.float32),
            jax.ShapeDtypeStruct((1, NB), jnp.int32),
            jax.ShapeDtypeStruct((1, NB), jnp.int32),
            jax.ShapeDtypeStruct((1, NB), jnp.int32),
            jax.ShapeDtypeStruct((1, 1), jnp.int32),
        ],
    )(x2, Wg, bg2)

    ys = pl.pallas_call(
        _ffn_body,
        grid_spec=pltpu.PrefetchScalarGridSpec(
            num_scalar_prefetch=5,
            grid=(NB,),
            in_specs=[
                pl.BlockSpec((S, D), lambda i, *_: (0, 0)),
                pl.BlockSpec((1, 1, F), lambda i, be, rd, nx, pm, pt:
                             (be[i], 0, 0)),
                pl.BlockSpec((1, 1, D), lambda i, be, rd, nx, pm, pt:
                             (be[i], 0, 0)),
                pl.BlockSpec(memory_space=pl.ANY),
                pl.BlockSpec(memory_space=pl.ANY),
            ],
            out_specs=pl.BlockSpec((BLK, D), lambda i, *_: (i, 0)),
            scratch_shapes=[
                pltpu.VMEM((BLK, D), jnp.float32),
                pltpu.VMEM((2, D, F), jnp.float32),
                pltpu.VMEM((2, F, D), jnp.float32),
                pltpu.SemaphoreType.DMA((2,)),
                pltpu.SemaphoreType.DMA((2,)),
            ],
        ),
        out_shape=jax.ShapeDtypeStruct((CAP, D), jnp.float32),
    )(be.reshape(NB), rid.reshape(NB), nxe.reshape(NB), perm.reshape(CAP),
      ptot.reshape(1), x2, b13, b23, W1, W2)

    out = pl.pallas_call(
        _combine_body,
        grid_spec=pltpu.PrefetchScalarGridSpec(
            num_scalar_prefetch=4,
            grid=(S // CBLK,),
            in_specs=[pl.BlockSpec((CAP, D), lambda i, *_: (0, 0))],
            out_specs=pl.BlockSpec((CBLK, D), lambda i, *_: (i, 0)),
        ),
        out_shape=jax.ShapeDtypeStruct((S, D), jnp.float32),
    )(posab[:, 0], posab[:, 1], wab[:, 0], wab[:, 1], ys)
    return out


def kernel(x, Wg, bg, W1, b1, W2, b2):
    out = _moe(x.reshape(S, D), Wg, bg.reshape(1, E), W1,
               b1.reshape(E, 1, F), W2, b2.reshape(E, 1, D))
    return out.reshape(B, S, D)
